# baseline (speedup 1.0000x reference)
import functools
import math

import jax
import jax.numpy as jnp
from jax import lax
from jax.experimental import pallas as pl
from jax.experimental.pallas import tpu as pltpu
from jax.experimental.pallas import tpu_sc as plsc

D_MODEL = 64
NUM_CORES = 2
NUM_SUBCORES = 16
NUM_WORKERS = NUM_CORES * NUM_SUBCORES
LANES = 16
CHUNK = 512
SCALE = math.sqrt(D_MODEL)


def kernel(x, table):
    batch = x.size
    per_worker = batch // NUM_WORKERS
    n_chunks = per_worker // CHUNK
    assert per_worker % CHUNK == 0
    xf = x.reshape(batch).astype(jnp.int32)

    mesh = plsc.VectorSubcoreMesh(core_axis_name="c", subcore_axis_name="s")

    @functools.partial(
        pl.kernel,
        mesh=mesh,
        out_type=jax.ShapeDtypeStruct((batch, D_MODEL), jnp.float32),
        compiler_params=pltpu.CompilerParams(use_tc_tiling_on_sc=False),
        scratch_types=[
            pltpu.VMEM((CHUNK,), jnp.int32),
            pltpu.VMEM((CHUNK, D_MODEL), jnp.float32),
            pltpu.SemaphoreType.DMA,
        ],
    )
    def gather_scale(table_hbm, idx_hbm, out_hbm, idx_v, rows_v, sem):
        wid = lax.axis_index("s") * NUM_CORES + lax.axis_index("c")
        base = wid * per_worker

        @pl.loop(0, n_chunks)
        def _(g):
            off = base + g * CHUNK
            pltpu.sync_copy(idx_hbm.at[pl.ds(off, CHUNK)], idx_v)
            pltpu.async_copy(table_hbm.at[idx_v], rows_v, sem).wait()

            @pl.loop(0, CHUNK)
            def _(r):
                @pl.loop(0, D_MODEL, step=LANES)
                def _(c):
                    slc = (pl.ds(r, 1), pl.ds(c, LANES))
                    rows_v.at[*slc][...] = rows_v.at[*slc][...] * SCALE

            pltpu.sync_copy(rows_v, out_hbm.at[pl.ds(off, CHUNK)])

    out = gather_scale(table, xf)
    return out.reshape(x.shape + (D_MODEL,))


# speedup vs baseline: 1.1319x; 1.1319x over previous
import functools
import math

import jax
import jax.numpy as jnp
from jax import lax
from jax.experimental import pallas as pl
from jax.experimental.pallas import tpu as pltpu
from jax.experimental.pallas import tpu_sc as plsc

D_MODEL = 64
NUM_CORES = 2
NUM_SUBCORES = 16
NUM_WORKERS = NUM_CORES * NUM_SUBCORES
LANES = 16
CHUNK = 320
NBUF = 4
SCALE = math.sqrt(D_MODEL)


def kernel(x, table):
    batch = x.size
    per_worker = batch // NUM_WORKERS
    n_chunks = per_worker // CHUNK
    assert per_worker % CHUNK == 0 and n_chunks % NBUF == 0 and n_chunks >= 8
    rounds = (n_chunks - 4) // NBUF
    xf = x.reshape(batch).astype(jnp.int32)

    mesh = plsc.VectorSubcoreMesh(core_axis_name="c", subcore_axis_name="s")

    @functools.partial(
        pl.kernel,
        mesh=mesh,
        out_type=jax.ShapeDtypeStruct((batch, D_MODEL), jnp.float32),
        compiler_params=pltpu.CompilerParams(use_tc_tiling_on_sc=False),
        scratch_types=[
            pltpu.VMEM((per_worker,), jnp.int32),
            pltpu.VMEM((NBUF * CHUNK, D_MODEL), jnp.float32),
            pltpu.SemaphoreType.DMA((NBUF,)),
            pltpu.SemaphoreType.DMA((NBUF,)),
        ],
    )
    def gather_scale(table_hbm, idx_hbm, out_hbm, idx_v, rows_v, gsem, ssem):
        wid = lax.axis_index("s") * NUM_CORES + lax.axis_index("c")
        base = wid * per_worker

        pltpu.sync_copy(idx_hbm.at[pl.ds(base, per_worker)], idx_v)

        def rows(b):
            return rows_v.at[pl.ds(b * CHUNK, CHUNK)]

        def gather(g, b):
            src = table_hbm.at[idx_v.at[pl.ds(g * CHUNK, CHUNK)]]
            return pltpu.make_async_copy(src, rows(b), gsem.at[b])

        def store(g, b):
            dst = out_hbm.at[pl.ds(base + g * CHUNK, CHUNK)]
            return pltpu.make_async_copy(rows(b), dst, ssem.at[b])

        def scale(b):
            buf = rows(b)

            @pl.loop(0, CHUNK)
            def _(r):
                for c in range(0, D_MODEL, LANES):
                    slc = (pl.ds(r, 1), pl.ds(c, LANES))
                    buf.at[*slc][...] = buf.at[*slc][...] * SCALE

        gather(0, 0).start()
        gather(1, 1).start()
        for p in range(2):
            gather(p, p).wait()
            scale(p)
            store(p, p).start()
            gather(p + 2, p + 2).start()

        @pl.loop(0, rounds)
        def _(r):
            for j in range(NBUF):
                b = (2 + j) % NBUF
                g = NBUF * r + 2 + j
                gather(g, b).wait()
                scale(b)
                store(g, b).start()
                pb = (j + 4) % NBUF
                store(g - 2, pb).wait()
                gather(g + 2, pb).start()

        for j in range(2):
            g = n_chunks - 2 + j
            b = g % NBUF
            gather(g, b).wait()
            scale(b)
            store(g, b).start()
        for b in range(NBUF):
            g = n_chunks - NBUF + b
            store(g, b).wait()

    out = gather_scale(table, xf)
    return out.reshape(x.shape + (D_MODEL,))
